# baseline (speedup 1.0000x reference)
import functools

import jax
import jax.numpy as jnp
from jax import lax
from jax.experimental import pallas as pl
from jax.experimental.pallas import tpu as pltpu
from jax.experimental.pallas import tpu_sc as plsc

B = 16
T = 50
P = 8
H = 64
W = 64
N = B * T
NC = 2
NS = 16
NWORK = NC * NS
RPW = N // NWORK


def _sc_body(xd_h, yd_h, dx_h, dy_h, ox_h, oy_h, z_h, out_h,
             xv, yv, dxv, dyv, oxv, oyv, buf0, buf1, buf2, buf3,
             sem0, sem1, sem2, sem3):
    w = lax.axis_index("s") * NC + lax.axis_index("c")
    base = w * RPW

    pltpu.sync_copy(xd_h.at[pl.ds(base * 16, RPW * 16)], xv)
    pltpu.sync_copy(yd_h.at[pl.ds(base * 16, RPW * 16)], yv)
    pltpu.sync_copy(dx_h.at[pl.ds(base * 16, RPW * 16)], dxv)
    pltpu.sync_copy(dy_h.at[pl.ds(base * 16, RPW * 16)], dyv)
    pltpu.sync_copy(ox_h.at[pl.ds(base * 16, RPW * 16)], oxv)
    pltpu.sync_copy(oy_h.at[pl.ds(base * 16, RPW * 16)], oyv)
    pltpu.sync_copy(z_h, buf0)
    pltpu.sync_copy(z_h, buf1)
    pltpu.sync_copy(z_h, buf2)
    pltpu.sync_copy(z_h, buf3)

    lane = lax.iota(jnp.int32, 16)
    lane_p = lane & 7
    mask_lo = lane < 8
    ones = jnp.full((16,), 1.0, jnp.float32)
    zeros_v = jnp.zeros((16,), jnp.float32)

    bufs = (buf0, buf1, buf2, buf3)
    sems = (sem0, sem1, sem2, sem3)
    prev = [None, None, None, None]
    handles = [None] * (4 * RPW)
    ok = riq = ciq = None
    QH = H // 4
    for hs in range(4 * RPW):
        s, q = hs >> 2, hs & 3
        b = hs & 3
        buf = bufs[b]
        if hs >= 4:
            handles[hs - 4].wait()
            idx_old, msk_old = prev[b]
            plsc.store_scatter(buf, idx_old, zeros_v, mask=msk_old)
        if q == 0:
            sl = pl.ds(s * 16, 16)
            cf = xv[sl] / dxv[sl] + oxv[sl]
            rf = yv[sl] / dyv[sl] + oyv[sl]
            ci = cf.astype(jnp.int32)
            ri = rf.astype(jnp.int32)
            ok = mask_lo & (ci >= 0) & (ci < W) & (ri >= 0) & (ri < H)
            ciq = jnp.clip(ci, 0, W - 1)
            riq = jnp.clip(ri, 0, H - 1)
        okq = ok & ((riq >= q * QH) & (riq < (q + 1) * QH))
        rloc = jnp.clip(riq - q * QH, 0, QH - 1)
        idx = [rloc, lane_p, ciq]
        plsc.store_scatter(buf, idx, ones, mask=okq)
        handles[hs] = pltpu.async_copy(
            buf, out_h.at[base + s, pl.ds(q * QH, QH)], sems[b])
        prev[b] = (idx, okq)
    for k in range(4):
        handles[4 * RPW - 4 + k].wait()


_sc_fn = functools.partial(
    pl.kernel,
    out_type=jax.ShapeDtypeStruct((N, H, P, W), jnp.float32),
    mesh=plsc.VectorSubcoreMesh(core_axis_name="c", subcore_axis_name="s"),
    compiler_params=pltpu.CompilerParams(needs_layout_passes=False),
    scratch_types=[
        pltpu.VMEM((RPW * 16,), jnp.float32),
        pltpu.VMEM((RPW * 16,), jnp.float32),
        pltpu.VMEM((RPW * 16,), jnp.float32),
        pltpu.VMEM((RPW * 16,), jnp.float32),
        pltpu.VMEM((RPW * 16,), jnp.float32),
        pltpu.VMEM((RPW * 16,), jnp.float32),
        pltpu.VMEM((H // 4, P, W), jnp.float32),
        pltpu.VMEM((H // 4, P, W), jnp.float32),
        pltpu.VMEM((H // 4, P, W), jnp.float32),
        pltpu.VMEM((H // 4, P, W), jnp.float32),
        pltpu.SemaphoreType.DMA,
        pltpu.SemaphoreType.DMA,
        pltpu.SemaphoreType.DMA,
        pltpu.SemaphoreType.DMA,
    ],
)(_sc_body)


def kernel(x, resolution, origin):
    pts = x.reshape(N, P, 2)
    xd = jnp.tile(pts[:, :, 0], (1, 2)).reshape(-1)
    yd = jnp.tile(pts[:, :, 1], (1, 2)).reshape(-1)
    res = resolution.reshape(N, 2)
    org = origin.reshape(N, 2)
    dx = jnp.tile(res[:, 0:1], (1, 16)).reshape(-1)
    dy = jnp.tile(res[:, 1:2], (1, 16)).reshape(-1)
    ox = jnp.tile(org[:, 1:2], (1, 16)).reshape(-1)
    oy = jnp.tile(org[:, 0:1], (1, 16)).reshape(-1)
    z = jnp.zeros((H // 4, P, W), jnp.float32)

    out = _sc_fn(xd, yd, dx, dy, ox, oy, z)
    out5 = out.reshape(B, T, H, P, W)
    return jnp.transpose(out5, (0, 1, 2, 4, 3))
